# baseline (speedup 1.0000x reference)
import functools

import jax
import jax.numpy as jnp
from jax import lax
from jax.experimental import pallas as pl
from jax.experimental.pallas import tpu as pltpu
from jax.experimental.pallas import tpu_sc as plsc

_DIM = 1024
_ROWS = 8192
_NC, _NS, _L = 2, 16, 16
_NW = _NC * _NS
_ROWS_W = _ROWS // _NW
_CROWS = 32
_NCHUNK = _ROWS_W // _CROWS
_NBUF = 3
_SCALE = float(_DIM) ** -0.5

_mesh = plsc.VectorSubcoreMesh(
    core_axis_name="c", subcore_axis_name="s",
    num_cores=_NC, num_subcores=_NS)


@functools.partial(
    pl.kernel,
    out_type=jax.ShapeDtypeStruct((_ROWS, _DIM), jnp.float32),
    mesh=_mesh,
    scratch_types=[
        [pltpu.VMEM((_CROWS, _DIM), jnp.float32)] * _NBUF,
        [pltpu.SemaphoreType.DMA] * _NBUF,
        [pltpu.SemaphoreType.DMA] * _NBUF,
    ],
)
def _scaled_copy(emb_hbm, out_hbm, bufs, sems_in, sems_out):
    wid = lax.axis_index("s") * _NC + lax.axis_index("c")
    base = wid * _ROWS_W

    def in_copy(c, b):
        return pltpu.make_async_copy(
            emb_hbm.at[pl.ds(base + c * _CROWS, _CROWS), :], bufs[b],
            sems_in[b])

    def out_copy(c, b):
        return pltpu.make_async_copy(
            bufs[b], out_hbm.at[pl.ds(base + c * _CROWS, _CROWS), :],
            sems_out[b])

    for b in range(min(_NBUF, _NCHUNK)):
        in_copy(b, b).start()

    for c in range(_NCHUNK):
        b = c % _NBUF
        if c >= 1 and c - 1 + _NBUF < _NCHUNK:
            pb = (c - 1) % _NBUF
            out_copy(c - 1, pb).wait()
            in_copy(c - 1 + _NBUF, pb).start()

        in_copy(c, b).wait()

        @plsc.parallel_loop(0, _CROWS * (_DIM // _L), unroll=16)
        def _scale(i, _buf=bufs[b]):
            r = i // (_DIM // _L)
            s = pl.ds((i % (_DIM // _L)) * _L, _L)
            _buf[r, s] = _buf[r, s] * _SCALE

        out_copy(c, b).start()

    for c in range(max(_NCHUNK - _NBUF, 0), _NCHUNK):
        out_copy(c, c % _NBUF).wait()


def kernel(x, emb):
    del x
    return _scaled_copy(emb)


# speedup vs baseline: 1.0305x; 1.0305x over previous
import functools

import jax
import jax.numpy as jnp
from jax import lax
from jax.experimental import pallas as pl
from jax.experimental.pallas import tpu as pltpu
from jax.experimental.pallas import tpu_sc as plsc

_DIM = 1024
_ROWS = 8192
_NC, _NS, _L = 2, 16, 16
_NW = _NC * _NS
_ROWS_W = _ROWS // _NW
_CROWS = 16
_NCHUNK = _ROWS_W // _CROWS
_NBUF = 4
_NROUND = _NCHUNK // _NBUF
_SCALE = float(_DIM) ** -0.5

_mesh = plsc.VectorSubcoreMesh(
    core_axis_name="c", subcore_axis_name="s",
    num_cores=_NC, num_subcores=_NS)


@functools.partial(
    pl.kernel,
    out_type=jax.ShapeDtypeStruct((_ROWS, _DIM), jnp.float32),
    mesh=_mesh,
    scratch_types=[
        [pltpu.VMEM((_CROWS, _DIM), jnp.float32)] * _NBUF,
        [pltpu.SemaphoreType.DMA] * _NBUF,
        [pltpu.SemaphoreType.DMA] * _NBUF,
    ],
)
def _scaled_copy(emb_hbm, out_hbm, bufs, sems_in, sems_out):
    wid = lax.axis_index("s") * _NC + lax.axis_index("c")
    base = wid * _ROWS_W

    def in_copy(c, b):
        return pltpu.make_async_copy(
            emb_hbm.at[pl.ds(base + c * _CROWS, _CROWS), :], bufs[b],
            sems_in[b])

    def out_copy(c, b):
        return pltpu.make_async_copy(
            bufs[b], out_hbm.at[pl.ds(base + c * _CROWS, _CROWS), :],
            sems_out[b])

    for b in range(_NBUF):
        in_copy(b, b).start()

    @pl.loop(0, _NROUND)
    def _round(g):
        for b in range(_NBUF):
            c = g * _NBUF + b
            pb = (b - 1) % _NBUF
            pc = c - 1

            @pl.when(jnp.logical_and(pc >= 0, pc + _NBUF < _NCHUNK))
            def _refill():
                out_copy(pc, pb).wait()
                in_copy(pc + _NBUF, pb).start()

            in_copy(c, b).wait()

            @plsc.parallel_loop(0, _CROWS * (_DIM // _L), unroll=16)
            def _scale(i, _buf=bufs[b]):
                r = i // (_DIM // _L)
                s = pl.ds((i % (_DIM // _L)) * _L, _L)
                _buf[r, s] = _buf[r, s] * _SCALE

            out_copy(c, b).start()

    for b in range(_NBUF):
        out_copy(_NCHUNK - _NBUF + b, b).wait()


def kernel(x, emb):
    del x
    return _scaled_copy(emb)


# speedup vs baseline: 1.0510x; 1.0199x over previous
import functools

import jax
import jax.numpy as jnp
from jax import lax
from jax.experimental import pallas as pl
from jax.experimental.pallas import tpu as pltpu
from jax.experimental.pallas import tpu_sc as plsc

_DIM = 1024
_ROWS = 8192
_NC, _NS, _L = 2, 16, 16
_NW = _NC * _NS
_ROWS_W = _ROWS // _NW
_CROWS = 8
_NCHUNK = _ROWS_W // _CROWS
_NBUF = 8
_NROUND = _NCHUNK // _NBUF
_LAG = 4
_SCALE = float(_DIM) ** -0.5

_mesh = plsc.VectorSubcoreMesh(
    core_axis_name="c", subcore_axis_name="s",
    num_cores=_NC, num_subcores=_NS)


@functools.partial(
    pl.kernel,
    out_type=jax.ShapeDtypeStruct((_ROWS, _DIM), jnp.float32),
    mesh=_mesh,
    scratch_types=[
        [pltpu.VMEM((_CROWS, _DIM), jnp.float32)] * _NBUF,
        [pltpu.SemaphoreType.DMA] * _NBUF,
        [pltpu.SemaphoreType.DMA] * _NBUF,
    ],
)
def _scaled_copy(emb_hbm, out_hbm, bufs, sems_in, sems_out):
    wid = lax.axis_index("s") * _NC + lax.axis_index("c")
    base = wid * _ROWS_W

    def in_copy(c, b):
        return pltpu.make_async_copy(
            emb_hbm.at[pl.ds(base + c * _CROWS, _CROWS), :], bufs[b],
            sems_in[b])

    def out_copy(c, b):
        return pltpu.make_async_copy(
            bufs[b], out_hbm.at[pl.ds(base + c * _CROWS, _CROWS), :],
            sems_out[b])

    for b in range(_NBUF):
        in_copy(b, b).start()

    @pl.loop(0, _NROUND)
    def _round(g):
        for b in range(_NBUF):
            c = g * _NBUF + b
            pb = (b - _LAG) % _NBUF
            pc = c - _LAG

            @pl.when(jnp.logical_and(pc >= 0, pc + _NBUF < _NCHUNK))
            def _refill():
                out_copy(pc, pb).wait()
                in_copy(pc + _NBUF, pb).start()

            in_copy(c, b).wait()

            @plsc.parallel_loop(0, _CROWS * (_DIM // _L), unroll=8)
            def _scale(i, _buf=bufs[b]):
                r = i // (_DIM // _L)
                s = pl.ds((i % (_DIM // _L)) * _L, _L)
                _buf[r, s] = _buf[r, s] * _SCALE

            out_copy(c, b).start()

    for b in range(_NBUF):
        out_copy(_NCHUNK - _NBUF + b, b).wait()


def kernel(x, emb):
    del x
    return _scaled_copy(emb)


# speedup vs baseline: 1.0706x; 1.0186x over previous
import functools

import jax
import jax.numpy as jnp
from jax import lax
from jax.experimental import pallas as pl
from jax.experimental.pallas import tpu as pltpu
from jax.experimental.pallas import tpu_sc as plsc

_DIM = 1024
_ROWS = 8192
_NC, _NS, _L = 2, 16, 16
_NW = _NC * _NS
_ROWS_W = _ROWS // _NW
_CROWS = 8
_NCHUNK = _ROWS_W // _CROWS
_NBUF = 8
_NROUND = _NCHUNK // _NBUF
_SCALE = float(_DIM) ** -0.5

_mesh = plsc.VectorSubcoreMesh(
    core_axis_name="c", subcore_axis_name="s",
    num_cores=_NC, num_subcores=_NS)


@functools.partial(
    pl.kernel,
    out_type=jax.ShapeDtypeStruct((_ROWS, _DIM), jnp.float32),
    mesh=_mesh,
    scratch_types=[
        [pltpu.VMEM((_CROWS, _DIM), jnp.float32)] * _NBUF,
        [pltpu.SemaphoreType.DMA] * _NBUF,
        [pltpu.SemaphoreType.DMA] * _NBUF,
    ],
)
def _scaled_copy(emb_hbm, out_hbm, bufs, sems_in, sems_out):
    wid = lax.axis_index("s") * _NC + lax.axis_index("c")
    base = wid * _ROWS_W

    def in_copy(c, b):
        return pltpu.make_async_copy(
            emb_hbm.at[pl.ds(base + c * _CROWS, _CROWS), :], bufs[b],
            sems_in[b])

    def out_copy(c, b):
        return pltpu.make_async_copy(
            bufs[b], out_hbm.at[pl.ds(base + c * _CROWS, _CROWS), :],
            sems_out[b])

    for b in range(_NBUF):
        in_copy(b, b).start()

    @pl.loop(0, _NROUND)
    def _round(g):
        for b in range(_NBUF):
            c = g * _NBUF + b
            pb = (b - 1) % _NBUF
            pc = c - 1

            @pl.when(jnp.logical_and(pc >= 0, pc + _NBUF < _NCHUNK))
            def _refill():
                out_copy(pc, pb).wait()
                in_copy(pc + _NBUF, pb).start()

            in_copy(c, b).wait()

            @plsc.parallel_loop(0, _CROWS * (_DIM // _L), unroll=8)
            def _scale(i, _buf=bufs[b]):
                r = i // (_DIM // _L)
                s = pl.ds((i % (_DIM // _L)) * _L, _L)
                _buf[r, s] = _buf[r, s] * _SCALE

            out_copy(c, b).start()

    for b in range(_NBUF):
        out_copy(_NCHUNK - _NBUF + b, b).wait()


def kernel(x, emb):
    del x
    return _scaled_copy(emb)
